# Initial kernel scaffold; baseline (speedup 1.0000x reference)
import jax
import jax.numpy as jnp
from jax.experimental import pallas as pl


def kernel(dist, embedding_table):
    raise NotImplementedError("write your pallas kernel here")


# speedup vs baseline: 7.8126x; 7.8126x over previous
import functools

import jax
import jax.numpy as jnp
from jax import lax
from jax.experimental import pallas as pl
from jax.experimental.pallas import tpu as pltpu
from jax.experimental.pallas import tpu_sc as plsc

NUM_HEADS = 16
B_TOTAL = 8 * 512 * 512
NW = 32
B_W = B_TOTAL // NW
CHUNK = 2048
N_CHUNKS = B_W // CHUNK

_mesh = plsc.VectorSubcoreMesh(core_axis_name="c", subcore_axis_name="s")


@functools.partial(
    pl.kernel,
    mesh=_mesh,
    out_type=jax.ShapeDtypeStruct((B_TOTAL, NUM_HEADS), jnp.float32),
    scratch_types=[
        pltpu.VMEM((CHUNK,), jnp.int32),
        pltpu.VMEM((CHUNK, NUM_HEADS), jnp.float32),
        pltpu.SemaphoreType.DMA,
    ],
    compiler_params=pltpu.CompilerParams(use_tc_tiling_on_sc=False),
)
def _gather_kernel(table_hbm, idx_hbm, out_hbm, idx_v, rows_v, sem):
    wid = lax.axis_index("s") * 2 + lax.axis_index("c")
    base = wid * B_W

    def body(g, carry):
        off = base + g * CHUNK
        pltpu.sync_copy(idx_hbm.at[pl.ds(off, CHUNK)], idx_v)
        pltpu.async_copy(table_hbm.at[idx_v], rows_v, sem).wait()
        pltpu.sync_copy(rows_v, out_hbm.at[pl.ds(off, CHUNK)])
        return carry

    lax.fori_loop(0, N_CHUNKS, body, 0)


def kernel(dist, embedding_table):
    idx = dist.reshape(-1).astype(jnp.int32)
    out = _gather_kernel(embedding_table, idx)
    return out.reshape(*dist.shape, NUM_HEADS)


# speedup vs baseline: 8.9263x; 1.1426x over previous
import functools

import jax
import jax.numpy as jnp
from jax import lax
from jax.experimental import pallas as pl
from jax.experimental.pallas import tpu as pltpu
from jax.experimental.pallas import tpu_sc as plsc

NUM_HEADS = 16
B_TOTAL = 8 * 512 * 512
NW = 32
B_W = B_TOTAL // NW
CHUNK = 2048
N_CHUNKS = B_W // CHUNK

_mesh = plsc.VectorSubcoreMesh(core_axis_name="c", subcore_axis_name="s")


@functools.partial(
    pl.kernel,
    mesh=_mesh,
    out_type=jax.ShapeDtypeStruct((B_TOTAL, NUM_HEADS), jnp.float32),
    scratch_types=[
        pltpu.VMEM((CHUNK,), jnp.int32),
        pltpu.VMEM((CHUNK, NUM_HEADS), jnp.float32),
        pltpu.VMEM_SHARED((512, NUM_HEADS), jnp.float32),
        pltpu.SemaphoreType.DMA,
    ],
    compiler_params=pltpu.CompilerParams(use_tc_tiling_on_sc=False),
)
def _gather_kernel(table_hbm, idx_hbm, out_hbm, idx_v, rows_v, table_v, sem):
    sid = lax.axis_index("s")
    wid = sid * 2 + lax.axis_index("c")
    base = wid * B_W

    @pl.when(sid == 0)
    def _stage_table():
        pltpu.sync_copy(table_hbm, table_v)

    plsc.subcore_barrier()

    def body(g, carry):
        off = base + g * CHUNK
        pltpu.sync_copy(idx_hbm.at[pl.ds(off, CHUNK)], idx_v)
        pltpu.async_copy(table_v.at[idx_v], rows_v, sem).wait()
        pltpu.sync_copy(rows_v, out_hbm.at[pl.ds(off, CHUNK)])
        return carry

    lax.fori_loop(0, N_CHUNKS, body, 0)


def kernel(dist, embedding_table):
    idx = dist.reshape(-1).astype(jnp.int32)
    out = _gather_kernel(embedding_table, idx)
    return out.reshape(*dist.shape, NUM_HEADS)


# speedup vs baseline: 9.5033x; 1.0646x over previous
import functools

import jax
import jax.numpy as jnp
from jax import lax
from jax.experimental import pallas as pl
from jax.experimental.pallas import tpu as pltpu
from jax.experimental.pallas import tpu_sc as plsc

NUM_HEADS = 16
VOCAB = 512
B_TOTAL = 8 * 512 * 512
NW = 32
B_W = B_TOTAL // NW
CHUNK = 1024
N_CHUNKS = B_W // CHUNK
NBUF = 4
N_ROUNDS = N_CHUNKS // NBUF

_mesh = plsc.VectorSubcoreMesh(core_axis_name="c", subcore_axis_name="s")


@functools.partial(
    pl.kernel,
    mesh=_mesh,
    out_type=jax.ShapeDtypeStruct((B_TOTAL, NUM_HEADS), jnp.float32),
    scratch_types=[
        pltpu.VMEM((NBUF, CHUNK), jnp.int32),
        pltpu.VMEM((NBUF, CHUNK, NUM_HEADS), jnp.float32),
        pltpu.VMEM_SHARED((VOCAB, NUM_HEADS), jnp.float32),
        pltpu.SemaphoreType.DMA((NBUF,)),
        pltpu.SemaphoreType.DMA((NBUF,)),
        pltpu.SemaphoreType.DMA((NBUF,)),
    ],
    compiler_params=pltpu.CompilerParams(use_tc_tiling_on_sc=False),
)
def _gather_kernel(table_hbm, idx_hbm, out_hbm, idx_v, rows_v, table_sh,
                   idx_sem, gat_sem, wb_sem):
    sid = lax.axis_index("s")
    wid = sid * 2 + lax.axis_index("c")
    base = wid * B_W

    @pl.when(sid == 0)
    def _stage_table():
        pltpu.sync_copy(table_hbm, table_sh)

    plsc.subcore_barrier()

    def start_idx(c, b):
        pltpu.async_copy(idx_hbm.at[pl.ds(base + c * CHUNK, CHUNK)],
                         idx_v.at[b], idx_sem.at[b])

    def wait_idx(b):
        pltpu.make_async_copy(idx_hbm.at[pl.ds(base, CHUNK)],
                              idx_v.at[b], idx_sem.at[b]).wait()

    def start_gather(b):
        pltpu.async_copy(table_sh.at[idx_v.at[b]], rows_v.at[b],
                         gat_sem.at[b])

    def wait_gather(b):
        pltpu.make_async_copy(table_sh.at[idx_v.at[b]], rows_v.at[b],
                              gat_sem.at[b]).wait()

    def start_wb(c, b):
        pltpu.async_copy(rows_v.at[b],
                         out_hbm.at[pl.ds(base + c * CHUNK, CHUNK)],
                         wb_sem.at[b])

    def wait_wb(b):
        pltpu.make_async_copy(rows_v.at[b],
                              out_hbm.at[pl.ds(base, CHUNK)],
                              wb_sem.at[b]).wait()

    for b in range(NBUF):
        start_idx(b, b)

    for g in range(NBUF):
        wait_idx(g)
        start_gather(g)
        if g >= 2:
            h = g - 2
            wait_gather(h)
            start_wb(h, h)
            start_idx(h + NBUF, h)

    def round_body(r, carry):
        g0 = r * NBUF
        for b in range(NBUF):
            bh = (b + 2) % NBUF
            wait_wb(b)
            wait_idx(b)
            start_gather(b)
            wait_gather(bh)
            start_wb(g0 + b - 2, bh)
            start_idx(g0 + b + 2, bh)
        return carry

    lax.fori_loop(1, N_ROUNDS - 1, round_body, 0)

    g0 = (N_ROUNDS - 1) * NBUF
    for b in range(NBUF):
        bh = (b + 2) % NBUF
        wait_wb(b)
        wait_idx(b)
        start_gather(b)
        wait_gather(bh)
        start_wb(g0 + b - 2, bh)
        if g0 + b + 2 < N_CHUNKS:
            start_idx(g0 + b + 2, bh)

    for g in (N_CHUNKS - 2, N_CHUNKS - 1):
        b = g % NBUF
        wait_gather(b)
        start_wb(g, b)
    for b in range(NBUF):
        wait_wb(b)


def kernel(dist, embedding_table):
    idx = dist.reshape(-1).astype(jnp.int32)
    out = _gather_kernel(embedding_table, idx)
    return out.reshape(*dist.shape, NUM_HEADS)
